# baseline (speedup 1.0000x reference)
import functools

import jax
import jax.numpy as jnp
from jax import lax
from jax.experimental import pallas as pl
from jax.experimental.pallas import tpu as pltpu
from jax.experimental.pallas import tpu_sc as plsc

BATCH = 16384
FIELDS = 26
DIM = 32
NUM_WORKERS = 32
BCHUNK = BATCH // NUM_WORKERS

_mesh = plsc.VectorSubcoreMesh(core_axis_name="c", subcore_axis_name="s")


@functools.partial(
    pl.kernel,
    mesh=_mesh,
    out_type=jax.ShapeDtypeStruct((FIELDS, DIM, BATCH), jnp.float32),
    scratch_types=[
        pltpu.VMEM((FIELDS, BCHUNK), jnp.int32),
        pltpu.VMEM((2, BCHUNK, DIM), jnp.float32),
        pltpu.VMEM((2, DIM, BCHUNK), jnp.float32),
        pltpu.SemaphoreType.DMA((2,)),
        pltpu.SemaphoreType.DMA((2,)),
    ],
    compiler_params=pltpu.CompilerParams(
        use_tc_tiling_on_sc=False, needs_layout_passes=False
    ),
)
def _gather_t(xt_hbm, table_hbm, out_hbm, idx_v, rows_v, tout_v, gsem, ssem):
    wid = lax.axis_index("s") * 2 + lax.axis_index("c")
    b0 = wid * BCHUNK

    pltpu.sync_copy(xt_hbm.at[:, pl.ds(b0, BCHUNK)], idx_v)

    lanes = lax.iota(jnp.int32, 16)

    def start_gather(f):
        b = f % 2
        return pltpu.async_copy(
            table_hbm.at[idx_v.at[f]], rows_v.at[b], gsem.at[b]
        )

    def transpose_block(b):
        def jbody(j, _):
            row_idx = lanes + j * 16
            for d in range(DIM):
                col_idx = jnp.full((16,), d, jnp.int32)
                v = plsc.load_gather(rows_v.at[b], [row_idx, col_idx])
                tout_v[b, d, pl.ds(j * 16, 16)] = v
            return _
        lax.fori_loop(0, BCHUNK // 16, jbody, 0)

    gathers = [None, None]
    stores = [None, None]
    gathers[0] = start_gather(0)
    for f in range(FIELDS):
        b = f % 2
        if f + 1 < FIELDS:
            gathers[1 - b] = start_gather(f + 1)
        gathers[b].wait()
        if stores[b] is not None:
            stores[b].wait()
        transpose_block(b)
        stores[b] = pltpu.async_copy(
            tout_v.at[b], out_hbm.at[f, :, pl.ds(b0, BCHUNK)], ssem.at[b]
        )
    stores[0].wait()
    stores[1].wait()


def kernel(x, table):
    out = _gather_t(x.T, table)
    return jnp.transpose(out, (2, 0, 1))


# speedup vs baseline: 1.1402x; 1.1402x over previous
import functools

import jax
import jax.numpy as jnp
from jax import lax
from jax.experimental import pallas as pl
from jax.experimental.pallas import tpu as pltpu
from jax.experimental.pallas import tpu_sc as plsc

BATCH = 16384
FIELDS = 26
DIM = 32
NUM_WORKERS = 32
BCHUNK = BATCH // NUM_WORKERS
NBUF = 3

_mesh = plsc.VectorSubcoreMesh(core_axis_name="c", subcore_axis_name="s")


@functools.partial(
    pl.kernel,
    mesh=_mesh,
    out_type=jax.ShapeDtypeStruct((BATCH, FIELDS, DIM), jnp.float32),
    scratch_types=[
        pltpu.VMEM((FIELDS, BCHUNK), jnp.int32),
        pltpu.VMEM((NBUF, BCHUNK, DIM), jnp.float32),
        pltpu.SemaphoreType.DMA((NBUF,)),
        pltpu.SemaphoreType.DMA((NBUF,)),
    ],
    compiler_params=pltpu.CompilerParams(
        use_tc_tiling_on_sc=False, needs_layout_passes=False
    ),
)
def _gather(xt_hbm, table_hbm, out_hbm, idx_v, rows_v, gsem, ssem):
    wid = lax.axis_index("s") * 2 + lax.axis_index("c")
    b0 = wid * BCHUNK

    pltpu.sync_copy(xt_hbm.at[:, pl.ds(b0, BCHUNK)], idx_v)

    def start_gather(f):
        b = f % NBUF
        return pltpu.async_copy(
            table_hbm.at[idx_v.at[f]], rows_v.at[b], gsem.at[b]
        )

    gathers = [None] * NBUF
    stores = [None] * NBUF
    for f in range(min(NBUF - 1, FIELDS)):
        gathers[f % NBUF] = start_gather(f)
    for f in range(FIELDS):
        b = f % NBUF
        nf = f + NBUF - 1
        if nf < FIELDS:
            nb = nf % NBUF
            if stores[nb] is not None:
                stores[nb].wait()
                stores[nb] = None
            gathers[nb] = start_gather(nf)
        gathers[b].wait()
        if stores[b] is not None:
            stores[b].wait()
        stores[b] = pltpu.async_copy(
            rows_v.at[b], out_hbm.at[pl.ds(b0, BCHUNK), f], ssem.at[b]
        )
    for s in stores:
        if s is not None:
            s.wait()


def kernel(x, table):
    return _gather(x.T, table)
